# baseline (speedup 1.0000x reference)
import functools

import jax
import jax.numpy as jnp
from jax import lax
from jax.experimental import pallas as pl
from jax.experimental.pallas import tpu as pltpu
from jax.experimental.pallas import tpu_sc as plsc

N = 10000
E = 320000
D = 128

NC = 2
NS = 16
NW = NC * NS
K = 128
TCH = E // K
CQ = TCH // NW
CR = TCH % NW


def _chunk_range(wid):
    start = wid * CQ + jnp.minimum(wid, CR)
    cnt = CQ + jnp.where(wid < CR, 1, 0)
    return start, cnt
RPT = 624
RPT_LAST = N - (NS - 1) * RPT


def _copy_tile_rows(s, copy_fn):
    base = s * RPT

    @pl.when(s < NS - 1)
    def _():
        copy_fn(base, RPT)

    @pl.when(s == NS - 1)
    def _():
        copy_fn(base, RPT_LAST)

_MESH = plsc.VectorSubcoreMesh(core_axis_name="c", subcore_axis_name="s")


@functools.partial(
    pl.kernel,
    out_type=jax.ShapeDtypeStruct((NC, N, D), jnp.float32),
    mesh=_MESH,
    scratch_types=(
        [pltpu.VMEM((K,), jnp.int32)] * 4
        + [pltpu.VMEM((K, D), jnp.float32)]
        + [pltpu.VMEM_SHARED((N, D), jnp.float32)]
        + [pltpu.SemaphoreType.DMA] * 8
    ),
)
def _sc_degree(dst_hbm, half_hbm, ones_hbm, out_hbm, *scr):
    didx = scr[0:4]
    ones_v = scr[4]
    acc = scr[5]
    isem = scr[6:10]
    ssem = scr[10:14]
    c = lax.axis_index("c")
    s = lax.axis_index("s")
    wid = c * NS + s
    cstart, cnt = _chunk_range(wid)

    def issue_idx(jc, b):
        pltpu.async_copy(dst_hbm.at[pl.ds((cstart + jc) * K, K)], didx[b],
                         isem[b])

    def wait_idx(b):
        pltpu.make_async_copy(dst_hbm.at[pl.ds(0, K)], didx[b],
                              isem[b]).wait()

    def issue_scatter(b):
        pltpu.async_copy(ones_v, acc.at[didx[b]], ssem[b], add=True)

    def wait_scatter(b):
        pltpu.make_async_copy(half_hbm.at[pl.ds(0, K), :], ones_v,
                              ssem[b]).wait()

    _copy_tile_rows(s, lambda b, n: pltpu.sync_copy(
        half_hbm.at[pl.ds(b, n), :], acc.at[pl.ds(b, n), :]))
    pltpu.sync_copy(ones_hbm, ones_v)
    plsc.subcore_barrier()

    for o in range(4):
        issue_idx(o, o)

    def body(j4, _):
        a = 4 * j4
        for o in range(4):
            @pl.when(a + o < cnt)
            def _(o=o):
                wait_idx(o)
                issue_scatter(o)

        for o in range(4):
            @pl.when(a + o + 4 < cnt)
            def _(o=o):
                wait_scatter(o)
                issue_idx(a + o + 4, o)

        return ()

    lax.fori_loop(0, (cnt + 3) // 4, body, (), unroll=False)
    for o in range(4):
        wait_scatter(o)
    plsc.subcore_barrier()
    _copy_tile_rows(s, lambda b, n: pltpu.sync_copy(
        acc.at[pl.ds(b, n), :], out_hbm.at[c, pl.ds(b, n), :]))


@functools.partial(
    pl.kernel,
    out_type=jax.ShapeDtypeStruct((NC, N, D), jnp.float32),
    mesh=_MESH,
    scratch_types=(
        [pltpu.VMEM((K,), jnp.int32)] * 12
        + [pltpu.VMEM((K, D), jnp.float32)] * 3
        + [pltpu.VMEM_SHARED((N, D), jnp.float32)]
        + [pltpu.SemaphoreType.DMA] * 12
    ),
)
def _sc_scatter(g_hbm, src_hbm, dst_hbm, out_hbm, *scr):
    sidx = scr[0:6]
    didx = scr[6:12]
    rows = scr[12:15]
    acc = scr[15]
    isem = scr[16:22]
    gsem = scr[22:25]
    ssem = scr[25:28]
    c = lax.axis_index("c")
    s = lax.axis_index("s")
    wid = c * NS + s
    cstart, cnt = _chunk_range(wid)

    def issue_idx(jc, b):
        base = (cstart + jc) * K
        pltpu.async_copy(src_hbm.at[pl.ds(base, K)], sidx[b], isem[b])
        pltpu.async_copy(dst_hbm.at[pl.ds(base, K)], didx[b], isem[b])

    def wait_idx(b):
        pltpu.make_async_copy(src_hbm.at[pl.ds(0, K)], sidx[b], isem[b]).wait()
        pltpu.make_async_copy(dst_hbm.at[pl.ds(0, K)], didx[b], isem[b]).wait()

    def issue_gather(ib, rb):
        pltpu.async_copy(g_hbm.at[sidx[ib]], rows[rb], gsem[rb])

    def wait_gather(rb):
        pltpu.make_async_copy(g_hbm.at[pl.ds(0, K), :], rows[rb],
                              gsem[rb]).wait()

    def issue_scatter(ib, rb):
        pltpu.async_copy(rows[rb], acc.at[didx[ib]], ssem[rb], add=True)

    def wait_scatter(rb):
        pltpu.make_async_copy(g_hbm.at[pl.ds(0, K), :], rows[rb],
                              ssem[rb]).wait()

    _copy_tile_rows(s, lambda b, n: pltpu.sync_copy(
        g_hbm.at[pl.ds(b, n), :], acc.at[pl.ds(b, n), :]))
    plsc.subcore_barrier()

    for o in range(6):
        issue_idx(o, o)
    for o in range(3):
        wait_idx(o)
        issue_gather(o, o)

    def _tick(a, ib):
        for o in range(3):
            @pl.when(a + o < cnt)
            def _(o=o):
                wait_gather(o)
                issue_scatter(ib + o, o)

        for o in range(3):
            @pl.when(a + o + 3 < cnt)
            def _(o=o):
                wait_scatter(o)
                wait_idx((ib + 3 + o) % 6)
                issue_gather((ib + 3 + o) % 6, o)

            @pl.when(a + o + 6 < cnt)
            def _(o=o):
                issue_idx(a + o + 6, (ib + o) % 6)

    def body(j6, _):
        a = 6 * j6
        _tick(a, 0)
        _tick(a + 3, 3)
        return ()

    lax.fori_loop(0, (cnt + 5) // 6, body, (), unroll=False)
    for o in range(3):
        wait_scatter(o)
    plsc.subcore_barrier()
    _copy_tile_rows(s, lambda b, n: pltpu.sync_copy(
        acc.at[pl.ds(b, n), :], out_hbm.at[c, pl.ds(b, n), :]))


def _tc1_body(x_ref, w1_ref, degp_ref, dinv_ref, g1_ref):
    deg = degp_ref[0, :, 0:1] + degp_ref[1, :, 0:1]
    dinv = lax.rsqrt(deg)
    h = jnp.dot(x_ref[...], w1_ref[...], preferred_element_type=jnp.float32)
    dinv_ref[...] = dinv
    g1_ref[...] = h * dinv


def _tc2_body(scatp_ref, g1_ref, dinv_ref, b1_ref, w2_ref, g2_ref):
    pre = scatp_ref[0] + scatp_ref[1] - g1_ref[...]
    h1 = jnp.maximum(dinv_ref[...] * pre + b1_ref[...], 0.0)
    h2 = jnp.dot(h1, w2_ref[...], preferred_element_type=jnp.float32)
    g2_ref[...] = h2 * dinv_ref[...]


def _tc3_body(scatp_ref, g2_ref, dinv_ref, b2_ref, gamma_ref, beta_ref,
              wf1_ref, bf1_ref, wf2_ref, bf2_ref, out_ref):
    pre = scatp_ref[0] + scatp_ref[1] - g2_ref[...]
    h = dinv_ref[...] * pre + b2_ref[...]
    h = jnp.where(h > 0, h, 0.01 * h)
    mu = jnp.mean(h, axis=0, keepdims=True)
    xc = h - mu
    var = jnp.mean(xc * xc, axis=0, keepdims=True)
    hn = gamma_ref[...] * xc / jnp.sqrt(var + 1e-5) + beta_ref[...]
    t = jnp.dot(hn, wf1_ref[...], preferred_element_type=jnp.float32)
    t = t + bf1_ref[...]
    t = jnp.where(t > 0, t, 0.01 * t)
    out_ref[...] = (jnp.dot(t, wf2_ref[...],
                            preferred_element_type=jnp.float32) + bf2_ref[...])


def kernel(x, edge_index, W1, b1, W2, b2, gamma, beta, Wf1, bf1, Wf2, bf2):
    f32 = jnp.float32
    src = edge_index[0]
    dst = edge_index[1]

    half = jnp.full((N, D), 0.5, f32)
    ones_blk = jnp.ones((K, D), f32)
    degp = _sc_degree(dst, half, ones_blk)

    dinv, g1 = pl.pallas_call(
        _tc1_body,
        out_shape=(jax.ShapeDtypeStruct((N, 1), f32),
                   jax.ShapeDtypeStruct((N, D), f32)),
    )(x, W1, degp)

    scatp1 = _sc_scatter(g1, src, dst)

    g2 = pl.pallas_call(
        _tc2_body,
        out_shape=jax.ShapeDtypeStruct((N, D), f32),
    )(scatp1, g1, dinv, b1.reshape(1, D), W2)

    scatp2 = _sc_scatter(g2, src, dst)

    Wf1p = jnp.zeros((D, 128), f32).at[:, :Wf1.shape[1]].set(Wf1)
    bf1p = jnp.zeros((1, 128), f32).at[0, :bf1.shape[0]].set(bf1)
    Wf2p = jnp.zeros((128, 128), f32).at[:Wf2.shape[0], :Wf2.shape[1]].set(Wf2)
    bf2p = jnp.zeros((1, 128), f32).at[0, :bf2.shape[0]].set(bf2)

    out128 = pl.pallas_call(
        _tc3_body,
        out_shape=jax.ShapeDtypeStruct((N, 128), f32),
    )(scatp2, g2, dinv, b2.reshape(1, D), gamma.reshape(1, D),
      beta.reshape(1, D), Wf1p, bf1p, Wf2p, bf2p)

    return out128[:, :Wf2.shape[1]]


# speedup vs baseline: 1.0024x; 1.0024x over previous
import functools

import jax
import jax.numpy as jnp
from jax import lax
from jax.experimental import pallas as pl
from jax.experimental.pallas import tpu as pltpu
from jax.experimental.pallas import tpu_sc as plsc

N = 10000
E = 320000
D = 128

NC = 2
NS = 16
NW = NC * NS
K = 128
TCH = E // K
CQ = TCH // NW
CR = TCH % NW


def _chunk_range(wid):
    start = wid * CQ + jnp.minimum(wid, CR)
    cnt = CQ + jnp.where(wid < CR, 1, 0)
    return start, cnt
RPT = 624
RPT_LAST = N - (NS - 1) * RPT


def _copy_tile_rows(s, copy_fn):
    base = s * RPT

    @pl.when(s < NS - 1)
    def _():
        copy_fn(base, RPT)

    @pl.when(s == NS - 1)
    def _():
        copy_fn(base, RPT_LAST)

_MESH = plsc.VectorSubcoreMesh(core_axis_name="c", subcore_axis_name="s")


@functools.partial(
    pl.kernel,
    out_type=jax.ShapeDtypeStruct((NC, N, D), jnp.float32),
    mesh=_MESH,
    scratch_types=(
        [pltpu.VMEM((K,), jnp.int32)] * 4
        + [pltpu.VMEM((K, D), jnp.float32)]
        + [pltpu.VMEM_SHARED((N, D), jnp.float32)]
        + [pltpu.SemaphoreType.DMA] * 8
    ),
)
def _sc_degree(eidx_hbm, zeros_hbm, ones_hbm, out_hbm, *scr):
    didx = scr[0:4]
    ones_v = scr[4]
    acc = scr[5]
    isem = scr[6:10]
    ssem = scr[10:14]
    c = lax.axis_index("c")
    s = lax.axis_index("s")
    wid = c * NS + s
    cstart, cnt = _chunk_range(wid)

    def issue_idx(jc, b):
        pltpu.async_copy(eidx_hbm.at[pl.ds(E + (cstart + jc) * K, K)],
                         didx[b], isem[b])

    def wait_idx(b):
        pltpu.make_async_copy(eidx_hbm.at[pl.ds(0, K)], didx[b],
                              isem[b]).wait()

    def issue_scatter(b):
        pltpu.async_copy(ones_v, acc.at[didx[b]], ssem[b], add=True)

    def wait_scatter(b):
        pltpu.make_async_copy(zeros_hbm, ones_v, ssem[b]).wait()

    def _init(b, n):
        for r in range(n // 128):
            pltpu.sync_copy(zeros_hbm, acc.at[pl.ds(b + r * 128, 128), :])
        rem = n % 128
        if rem:
            pltpu.sync_copy(zeros_hbm.at[pl.ds(0, rem), :],
                            acc.at[pl.ds(b + n - rem, rem), :])

    _copy_tile_rows(s, _init)
    pltpu.sync_copy(ones_hbm, ones_v)
    plsc.subcore_barrier()

    for o in range(4):
        issue_idx(o, o)

    def body(j4, _):
        a = 4 * j4
        for o in range(4):
            @pl.when(a + o < cnt)
            def _(o=o):
                wait_idx(o)
                issue_scatter(o)

        for o in range(4):
            @pl.when(a + o + 4 < cnt)
            def _(o=o):
                wait_scatter(o)
                issue_idx(a + o + 4, o)

        return ()

    lax.fori_loop(0, (cnt + 3) // 4, body, (), unroll=False)
    for o in range(4):
        wait_scatter(o)
    plsc.subcore_barrier()
    _copy_tile_rows(s, lambda b, n: pltpu.sync_copy(
        acc.at[pl.ds(b, n), :], out_hbm.at[c, pl.ds(b, n), :]))


@functools.partial(
    pl.kernel,
    out_type=jax.ShapeDtypeStruct((NC, N, D), jnp.float32),
    mesh=_MESH,
    scratch_types=(
        [pltpu.VMEM((K,), jnp.int32)] * 12
        + [pltpu.VMEM((K, D), jnp.float32)] * 3
        + [pltpu.VMEM_SHARED((N, D), jnp.float32)]
        + [pltpu.SemaphoreType.DMA] * 12
    ),
)
def _sc_scatter(g_hbm, eidx_hbm, out_hbm, *scr):
    sidx = scr[0:6]
    didx = scr[6:12]
    rows = scr[12:15]
    acc = scr[15]
    isem = scr[16:22]
    gsem = scr[22:25]
    ssem = scr[25:28]
    c = lax.axis_index("c")
    s = lax.axis_index("s")
    wid = c * NS + s
    cstart, cnt = _chunk_range(wid)

    def issue_idx(jc, b):
        base = (cstart + jc) * K
        pltpu.async_copy(eidx_hbm.at[pl.ds(base, K)], sidx[b], isem[b])
        pltpu.async_copy(eidx_hbm.at[pl.ds(E + base, K)], didx[b], isem[b])

    def wait_idx(b):
        pltpu.make_async_copy(eidx_hbm.at[pl.ds(0, K)], sidx[b],
                              isem[b]).wait()
        pltpu.make_async_copy(eidx_hbm.at[pl.ds(0, K)], didx[b],
                              isem[b]).wait()

    def issue_gather(ib, rb):
        pltpu.async_copy(g_hbm.at[sidx[ib]], rows[rb], gsem[rb])

    def wait_gather(rb):
        pltpu.make_async_copy(g_hbm.at[pl.ds(0, K), :], rows[rb],
                              gsem[rb]).wait()

    def issue_scatter(ib, rb):
        pltpu.async_copy(rows[rb], acc.at[didx[ib]], ssem[rb], add=True)

    def wait_scatter(rb):
        pltpu.make_async_copy(g_hbm.at[pl.ds(0, K), :], rows[rb],
                              ssem[rb]).wait()

    _copy_tile_rows(s, lambda b, n: pltpu.sync_copy(
        g_hbm.at[pl.ds(b, n), :], acc.at[pl.ds(b, n), :]))
    plsc.subcore_barrier()

    for o in range(6):
        issue_idx(o, o)
    for o in range(3):
        wait_idx(o)
        issue_gather(o, o)

    def _tick(a, ib):
        for o in range(3):
            @pl.when(a + o < cnt)
            def _(o=o):
                wait_gather(o)
                issue_scatter(ib + o, o)

        for o in range(3):
            @pl.when(a + o + 3 < cnt)
            def _(o=o):
                wait_scatter(o)
                wait_idx((ib + 3 + o) % 6)
                issue_gather((ib + 3 + o) % 6, o)

            @pl.when(a + o + 6 < cnt)
            def _(o=o):
                issue_idx(a + o + 6, (ib + o) % 6)

    def body(j6, _):
        a = 6 * j6
        _tick(a, 0)
        _tick(a + 3, 3)
        return ()

    lax.fori_loop(0, (cnt + 5) // 6, body, (), unroll=False)
    for o in range(3):
        wait_scatter(o)
    plsc.subcore_barrier()
    _copy_tile_rows(s, lambda b, n: pltpu.sync_copy(
        acc.at[pl.ds(b, n), :], out_hbm.at[c, pl.ds(b, n), :]))


def _tc_mm_body(x_ref, w1_ref, h_ref):
    h_ref[...] = jnp.dot(x_ref[...], w1_ref[...],
                         preferred_element_type=jnp.float32)


def _tc1_body(h_ref, degp_ref, dinv_ref, g1_ref):
    deg = 1.0 + degp_ref[0, :, 0:1] + degp_ref[1, :, 0:1]
    dinv = lax.rsqrt(deg)
    dinv_ref[...] = dinv
    g1_ref[...] = h_ref[...] * dinv


def _tc2_body(scatp_ref, g1_ref, dinv_ref, b1_ref, w2_ref, g2_ref):
    pre = scatp_ref[0] + scatp_ref[1] - g1_ref[...]
    h1 = jnp.maximum(dinv_ref[...] * pre + b1_ref[...], 0.0)
    h2 = jnp.dot(h1, w2_ref[...], preferred_element_type=jnp.float32)
    g2_ref[...] = h2 * dinv_ref[...]


def _tc3_body(scatp_ref, g2_ref, dinv_ref, b2_ref, gamma_ref, beta_ref,
              wf1_ref, bf1_ref, wf2_ref, bf2_ref, out_ref):
    pre = scatp_ref[0] + scatp_ref[1] - g2_ref[...]
    h = dinv_ref[...] * pre + b2_ref[...]
    h = jnp.where(h > 0, h, 0.01 * h)
    mu = jnp.mean(h, axis=0, keepdims=True)
    xc = h - mu
    var = jnp.mean(xc * xc, axis=0, keepdims=True)
    hn = gamma_ref[...] * xc / jnp.sqrt(var + 1e-5) + beta_ref[...]
    t = jnp.dot(hn, wf1_ref[...], preferred_element_type=jnp.float32)
    t = t + bf1_ref[...]
    t = jnp.where(t > 0, t, 0.01 * t)
    out_ref[...] = (jnp.dot(t, wf2_ref[...],
                            preferred_element_type=jnp.float32) + bf2_ref[...])


def kernel(x, edge_index, W1, b1, W2, b2, gamma, beta, Wf1, bf1, Wf2, bf2):
    f32 = jnp.float32
    eflat = edge_index.reshape(2 * E)

    zeros_blk = jnp.zeros((K, D), f32)
    ones_blk = jnp.ones((K, D), f32)
    degp = _sc_degree(eflat, zeros_blk, ones_blk)

    h1pre = pl.pallas_call(
        _tc_mm_body,
        out_shape=jax.ShapeDtypeStruct((N, D), f32),
    )(x, W1)

    dinv, g1 = pl.pallas_call(
        _tc1_body,
        out_shape=(jax.ShapeDtypeStruct((N, 1), f32),
                   jax.ShapeDtypeStruct((N, D), f32)),
    )(h1pre, degp)

    scatp1 = _sc_scatter(g1, eflat)

    g2 = pl.pallas_call(
        _tc2_body,
        out_shape=jax.ShapeDtypeStruct((N, D), f32),
    )(scatp1, g1, dinv, b1.reshape(1, D), W2)

    scatp2 = _sc_scatter(g2, eflat)

    Wf1p = jnp.zeros((D, 128), f32).at[:, :Wf1.shape[1]].set(Wf1)
    bf1p = jnp.zeros((1, 128), f32).at[0, :bf1.shape[0]].set(bf1)
    Wf2p = jnp.zeros((128, 128), f32).at[:Wf2.shape[0], :Wf2.shape[1]].set(Wf2)
    bf2p = jnp.zeros((1, 128), f32).at[0, :bf2.shape[0]].set(bf2)

    out128 = pl.pallas_call(
        _tc3_body,
        out_shape=jax.ShapeDtypeStruct((N, 128), f32),
    )(scatp2, g2, dinv, b2.reshape(1, D), gamma.reshape(1, D),
      beta.reshape(1, D), Wf1p, bf1p, Wf2p, bf2p)

    return out128[:, :Wf2.shape[1]]


# speedup vs baseline: 1.0183x; 1.0158x over previous
import functools

import jax
import jax.numpy as jnp
from jax import lax
from jax.experimental import pallas as pl
from jax.experimental.pallas import tpu as pltpu
from jax.experimental.pallas import tpu_sc as plsc

N = 10000
E = 320000
D = 128

NC = 2
NS = 16
NW = NC * NS
K = 128
TCH = E // K
CQ = TCH // NW
CR = TCH % NW


def _chunk_range(wid):
    start = wid * CQ + jnp.minimum(wid, CR)
    cnt = CQ + jnp.where(wid < CR, 1, 0)
    return start, cnt
RPT = 624
RPT_LAST = N - (NS - 1) * RPT


def _copy_tile_rows(s, copy_fn):
    base = s * RPT

    @pl.when(s < NS - 1)
    def _():
        copy_fn(base, RPT)

    @pl.when(s == NS - 1)
    def _():
        copy_fn(base, RPT_LAST)

_MESH = plsc.VectorSubcoreMesh(core_axis_name="c", subcore_axis_name="s")


@functools.partial(
    pl.kernel,
    out_type=jax.ShapeDtypeStruct((NC, N, D), jnp.float32),
    mesh=_MESH,
    scratch_types=(
        [pltpu.VMEM((K,), jnp.int32)] * 4
        + [pltpu.VMEM((K, D), jnp.float32)]
        + [pltpu.VMEM_SHARED((N, D), jnp.float32)]
        + [pltpu.SemaphoreType.DMA] * 8
    ),
)
def _sc_degree(eidx_hbm, zeros_hbm, ones_hbm, out_hbm, *scr):
    didx = scr[0:4]
    ones_v = scr[4]
    acc = scr[5]
    isem = scr[6:10]
    ssem = scr[10:14]
    c = lax.axis_index("c")
    s = lax.axis_index("s")
    wid = c * NS + s
    cstart, cnt = _chunk_range(wid)

    def issue_idx(jc, b):
        pltpu.async_copy(eidx_hbm.at[pl.ds(E + (cstart + jc) * K, K)],
                         didx[b], isem[b])

    def wait_idx(b):
        pltpu.make_async_copy(eidx_hbm.at[pl.ds(0, K)], didx[b],
                              isem[b]).wait()

    def issue_scatter(b):
        pltpu.async_copy(ones_v, acc.at[didx[b]], ssem[b], add=True)

    def wait_scatter(b):
        pltpu.make_async_copy(zeros_hbm.at[pl.ds(0, K), :], ones_v,
                              ssem[b]).wait()

    _copy_tile_rows(s, lambda b, n: pltpu.sync_copy(
        zeros_hbm.at[pl.ds(0, n), :], acc.at[pl.ds(b, n), :]))
    pltpu.sync_copy(ones_hbm, ones_v)
    plsc.subcore_barrier()

    for o in range(4):
        issue_idx(o, o)

    def body(j4, _):
        a = 4 * j4
        for o in range(4):
            @pl.when(a + o < cnt)
            def _(o=o):
                wait_idx(o)
                issue_scatter(o)

        for o in range(4):
            @pl.when(a + o + 4 < cnt)
            def _(o=o):
                wait_scatter(o)
                issue_idx(a + o + 4, o)

        return ()

    lax.fori_loop(0, (cnt + 3) // 4, body, (), unroll=False)
    for o in range(4):
        wait_scatter(o)
    plsc.subcore_barrier()
    _copy_tile_rows(s, lambda b, n: pltpu.sync_copy(
        acc.at[pl.ds(b, n), :], out_hbm.at[c, pl.ds(b, n), :]))


@functools.partial(
    pl.kernel,
    out_type=jax.ShapeDtypeStruct((NC, N, D), jnp.float32),
    mesh=_MESH,
    scratch_types=(
        [pltpu.VMEM((K,), jnp.int32)] * 12
        + [pltpu.VMEM((K, D), jnp.float32)] * 3
        + [pltpu.VMEM_SHARED((N, D), jnp.float32)]
        + [pltpu.SemaphoreType.DMA] * 12
    ),
)
def _sc_scatter(g_hbm, eidx_hbm, out_hbm, *scr):
    sidx = scr[0:6]
    didx = scr[6:12]
    rows = scr[12:15]
    acc = scr[15]
    isem = scr[16:22]
    gsem = scr[22:25]
    ssem = scr[25:28]
    c = lax.axis_index("c")
    s = lax.axis_index("s")
    wid = c * NS + s
    cstart, cnt = _chunk_range(wid)

    def issue_idx(jc, b):
        base = (cstart + jc) * K
        pltpu.async_copy(eidx_hbm.at[pl.ds(base, K)], sidx[b], isem[b])
        pltpu.async_copy(eidx_hbm.at[pl.ds(E + base, K)], didx[b], isem[b])

    def wait_idx(b):
        pltpu.make_async_copy(eidx_hbm.at[pl.ds(0, K)], sidx[b],
                              isem[b]).wait()
        pltpu.make_async_copy(eidx_hbm.at[pl.ds(0, K)], didx[b],
                              isem[b]).wait()

    def issue_gather(ib, rb):
        pltpu.async_copy(g_hbm.at[sidx[ib]], rows[rb], gsem[rb])

    def wait_gather(rb):
        pltpu.make_async_copy(g_hbm.at[pl.ds(0, K), :], rows[rb],
                              gsem[rb]).wait()

    def issue_scatter(ib, rb):
        pltpu.async_copy(rows[rb], acc.at[didx[ib]], ssem[rb], add=True)

    def wait_scatter(rb):
        pltpu.make_async_copy(g_hbm.at[pl.ds(0, K), :], rows[rb],
                              ssem[rb]).wait()

    _copy_tile_rows(s, lambda b, n: pltpu.sync_copy(
        g_hbm.at[pl.ds(b, n), :], acc.at[pl.ds(b, n), :]))
    plsc.subcore_barrier()

    for o in range(6):
        issue_idx(o, o)
    for o in range(3):
        wait_idx(o)
        issue_gather(o, o)

    def _tick(a, ib):
        for o in range(3):
            @pl.when(a + o < cnt)
            def _(o=o):
                wait_gather(o)
                issue_scatter(ib + o, o)

        for o in range(3):
            @pl.when(a + o + 3 < cnt)
            def _(o=o):
                wait_scatter(o)
                wait_idx((ib + 3 + o) % 6)
                issue_gather((ib + 3 + o) % 6, o)

            @pl.when(a + o + 6 < cnt)
            def _(o=o):
                issue_idx(a + o + 6, (ib + o) % 6)

    def body(j6, _):
        a = 6 * j6
        _tick(a, 0)
        _tick(a + 3, 3)
        return ()

    lax.fori_loop(0, (cnt + 5) // 6, body, (), unroll=False)
    for o in range(3):
        wait_scatter(o)
    plsc.subcore_barrier()
    _copy_tile_rows(s, lambda b, n: pltpu.sync_copy(
        acc.at[pl.ds(b, n), :], out_hbm.at[c, pl.ds(b, n), :]))


def _tc_mm_body(x_ref, w1_ref, h_ref):
    h_ref[...] = jnp.dot(x_ref[...], w1_ref[...],
                         preferred_element_type=jnp.float32)


def _tc1_body(h_ref, degp_ref, dinv_ref, g1_ref):
    deg = 1.0 + degp_ref[0, :, 0:1] + degp_ref[1, :, 0:1]
    dinv = lax.rsqrt(deg)
    dinv_ref[...] = dinv
    g1_ref[...] = h_ref[...] * dinv


def _tc2_body(scatp_ref, g1_ref, dinv_ref, b1_ref, w2_ref, g2_ref):
    pre = scatp_ref[0] + scatp_ref[1] - g1_ref[...]
    h1 = jnp.maximum(dinv_ref[...] * pre + b1_ref[...], 0.0)
    h2 = jnp.dot(h1, w2_ref[...], preferred_element_type=jnp.float32)
    g2_ref[...] = h2 * dinv_ref[...]


def _tc3_body(scatp_ref, g2_ref, dinv_ref, b2_ref, gamma_ref, beta_ref,
              wf1_ref, bf1_ref, wf2_ref, bf2_ref, out_ref):
    pre = scatp_ref[0] + scatp_ref[1] - g2_ref[...]
    h = dinv_ref[...] * pre + b2_ref[...]
    h = jnp.where(h > 0, h, 0.01 * h)
    mu = jnp.mean(h, axis=0, keepdims=True)
    xc = h - mu
    var = jnp.mean(xc * xc, axis=0, keepdims=True)
    hn = gamma_ref[...] * xc / jnp.sqrt(var + 1e-5) + beta_ref[...]
    t = jnp.dot(hn, wf1_ref[...], preferred_element_type=jnp.float32)
    t = t + bf1_ref[...]
    t = jnp.where(t > 0, t, 0.01 * t)
    res = (jnp.dot(t, wf2_ref[...],
                   preferred_element_type=jnp.float32) + bf2_ref[...])
    out_ref[...] = res[:, :8]


def kernel(x, edge_index, W1, b1, W2, b2, gamma, beta, Wf1, bf1, Wf2, bf2):
    f32 = jnp.float32
    eflat = edge_index.reshape(2 * E)

    zeros_blk = jnp.zeros((RPT_LAST, D), f32)
    ones_blk = jnp.ones((K, D), f32)
    degp = _sc_degree(eflat, zeros_blk, ones_blk)

    h1pre = pl.pallas_call(
        _tc_mm_body,
        out_shape=jax.ShapeDtypeStruct((N, D), f32),
    )(x, W1)

    dinv, g1 = pl.pallas_call(
        _tc1_body,
        out_shape=(jax.ShapeDtypeStruct((N, 1), f32),
                   jax.ShapeDtypeStruct((N, D), f32)),
    )(h1pre, degp)

    scatp1 = _sc_scatter(g1, eflat)

    g2 = pl.pallas_call(
        _tc2_body,
        out_shape=jax.ShapeDtypeStruct((N, D), f32),
    )(scatp1, g1, dinv, b1.reshape(1, D), W2)

    scatp2 = _sc_scatter(g2, eflat)

    Wf1p = jnp.zeros((D, 128), f32).at[:, :Wf1.shape[1]].set(Wf1)
    bf1p = jnp.zeros((1, 128), f32).at[0, :bf1.shape[0]].set(bf1)
    Wf2p = jnp.zeros((128, 128), f32).at[:Wf2.shape[0], :Wf2.shape[1]].set(Wf2)
    bf2p = jnp.zeros((1, 128), f32).at[0, :bf2.shape[0]].set(bf2)

    out8 = pl.pallas_call(
        _tc3_body,
        out_shape=jax.ShapeDtypeStruct((N, 8), f32),
    )(scatp2, g2, dinv, b2.reshape(1, D), gamma.reshape(1, D),
      beta.reshape(1, D), Wf1p, bf1p, Wf2p, bf2p)

    return out8[:, :Wf2.shape[1]]
